# Initial kernel scaffold; baseline (speedup 1.0000x reference)
import functools

import jax
import jax.numpy as jnp
from jax import lax
from jax.experimental import pallas as pl
from jax.experimental.pallas import tpu as pltpu
from jax.experimental.pallas import tpu_sc as plsc

N = 10000
E = 320000
H = 128
HH = H // 2
NC, NS, L = 2, 16, 16
EPT = E // NS
CH = 80
NCHUNK = EPT // CH
RPT = N // NS
RB = 125
NRB = RPT // RB


def _prep_kernel(x_ref, w_ref, t_ref, b_ref):
    x = x_ref[...]
    ab = jnp.dot(x, w_ref[...], preferred_element_type=jnp.float32)
    t_ref[0, :, :HH] = ab[:, :HH]
    t_ref[0, :, HH:] = x[:, :HH]
    t_ref[1, :, :HH] = ab[:, HH:H]
    t_ref[1, :, HH:] = x[:, HH:]
    b_ref[0, :, :] = ab[:, H:H + HH]
    b_ref[1, :, :] = ab[:, H + HH:]


def _prep(state, wcat):
    blk = 1000
    grid = N // blk
    t, b = pl.pallas_call(
        _prep_kernel,
        grid=(grid,),
        in_specs=[
            pl.BlockSpec((blk, H), lambda i: (i, 0)),
            pl.BlockSpec((H, 2 * H), lambda i: (0, 0)),
        ],
        out_specs=[
            pl.BlockSpec((2, blk, H), lambda i: (0, i, 0)),
            pl.BlockSpec((2, blk, HH), lambda i: (0, i, 0)),
        ],
        out_shape=[
            jax.ShapeDtypeStruct((2, N, H), jnp.float32),
            jax.ShapeDtypeStruct((2, N, HH), jnp.float32),
        ],
    )(state, wcat)
    return t.reshape(2 * N, H), b.reshape(2 * N, HH)


def _edge_body(t_hbm, b_hbm, src_hbm, dst_hbm, dist_hbm, out_hbm,
               si_v, bi_v, di_v, dv_v, gt_v, gb_v, pc_v, zb_v, acc_sh,
               sem1, sem2):
    cid = lax.axis_index("c")
    sid = lax.axis_index("s")
    coff = cid * N

    def _zrow(r, carry):
        for c in range(H // L):
            zb_v[r, pl.ds(c * L, L)] = jnp.zeros((L,), jnp.float32)
        return carry
    lax.fori_loop(0, RB, _zrow, 0)

    def _zcopy(bk, carry):
        pltpu.sync_copy(zb_v, acc_sh.at[pl.ds(sid * RPT + bk * RB, RB)])
        return carry
    lax.fori_loop(0, NRB, _zcopy, 0)
    plsc.subcore_barrier()

    def _chunk(k, carry):
        base = sid * EPT + k * CH
        pltpu.sync_copy(src_hbm.at[pl.ds(base, CH)], si_v)
        pltpu.sync_copy(dst_hbm.at[pl.ds(base, CH)], di_v)
        pltpu.sync_copy(dist_hbm.at[pl.ds(base, CH)], dv_v)

        def _off(i, c2):
            s = si_v[pl.ds(i * L, L)]
            si_v[pl.ds(i * L, L)] = s + coff
            d = di_v[pl.ds(i * L, L)]
            bi_v[pl.ds(i * L, L)] = d + coff
            return c2
        lax.fori_loop(0, CH // L, _off, 0)

        cp1 = pltpu.async_copy(t_hbm.at[si_v], gt_v, sem1)
        cp2 = pltpu.async_copy(b_hbm.at[bi_v], gb_v, sem2)
        cp1.wait()
        cp2.wait()

        def _edge(j, c2):
            d16 = plsc.load_gather(dv_v, [jnp.full((L,), j, jnp.int32)])
            for c in range(HH // L):
                a = gt_v[j, pl.ds(c * L, L)]
                b = gb_v[j, pl.ds(c * L, L)]
                s = gt_v[j, pl.ds(HH + c * L, L)]
                u = (a + b) * d16
                t = jnp.maximum(u, 0.01 * u)
                p = jnp.exp(t)
                pc_v[j, pl.ds(c * L, L)] = p * s
                pc_v[j, pl.ds(HH + c * L, L)] = p
            return c2
        lax.fori_loop(0, CH, _edge, 0)

        pltpu.sync_copy(pc_v, acc_sh.at[di_v], add=True)
        return carry
    lax.fori_loop(0, NCHUNK, _chunk, 0)
    plsc.subcore_barrier()

    def _wb(bk, carry):
        r0 = sid * RPT + bk * RB
        pltpu.sync_copy(acc_sh.at[pl.ds(r0, RB)], zb_v)
        pltpu.sync_copy(zb_v, out_hbm.at[pl.ds(coff + r0, RB)])
        return carry
    lax.fori_loop(0, NRB, _wb, 0)


def _edge_pass(t, b, src, dst, dist):
    mesh = plsc.VectorSubcoreMesh(
        core_axis_name="c", subcore_axis_name="s", num_cores=NC,
        num_subcores=NS)
    f = pl.kernel(
        _edge_body,
        out_type=jax.ShapeDtypeStruct((2 * N, H), jnp.float32),
        mesh=mesh,
        scratch_types=[
            pltpu.VMEM((CH,), jnp.int32),
            pltpu.VMEM((CH,), jnp.int32),
            pltpu.VMEM((CH,), jnp.int32),
            pltpu.VMEM((CH,), jnp.float32),
            pltpu.VMEM((CH, H), jnp.float32),
            pltpu.VMEM((CH, HH), jnp.float32),
            pltpu.VMEM((CH, H), jnp.float32),
            pltpu.VMEM((RB, H), jnp.float32),
            pltpu.VMEM_SHARED((N, H), jnp.float32),
            pltpu.SemaphoreType.DMA,
            pltpu.SemaphoreType.DMA,
        ],
    )
    return f(t, b, src, dst, dist)


def _combine_kernel(lo_ref, hi_ref, o_ref):
    lo = lo_ref[...]
    hi = hi_ref[...]
    n = jnp.concatenate([lo[:, :HH], hi[:, :HH]], axis=1)
    d = jnp.concatenate([lo[:, HH:], hi[:, HH:]], axis=1)
    o_ref[...] = jnp.maximum(n / (d + 1e-16), 0.0)


def _combine(nd):
    blk = 1000
    grid = N // blk
    return pl.pallas_call(
        _combine_kernel,
        grid=(grid,),
        in_specs=[
            pl.BlockSpec((blk, H), lambda i: (i, 0)),
            pl.BlockSpec((blk, H), lambda i: (i + grid, 0)),
        ],
        out_specs=pl.BlockSpec((blk, H), lambda i: (i, 0)),
        out_shape=jax.ShapeDtypeStruct((N, H), jnp.float32),
    )(nd, nd)


def kernel(state, feature, edge_index, edge_dist, W):
    src = edge_index[0]
    dst = edge_index[1]
    dist = edge_dist.reshape(-1)
    wcat = jnp.concatenate([W[:H, :], W[H:, :]], axis=1)
    t, b = _prep(state, wcat)
    nd = _edge_pass(t, b, src, dst, dist)
    return _combine(nd)


# speedup vs baseline: 2.8381x; 2.8381x over previous
import functools

import jax
import jax.numpy as jnp
from jax import lax
from jax.experimental import pallas as pl
from jax.experimental.pallas import tpu as pltpu
from jax.experimental.pallas import tpu_sc as plsc

N = 10000
NP = 10240
E = 320000
H = 128
HH = H // 2
NC, NS, L = 2, 16, 16
EPT = E // NS
CH = 80
NCHUNK = EPT // CH
RPT = NP // NS
RB = 128
NRB = RPT // RB


def _prep_kernel(x_ref, w_ref, t_ref, b_ref):
    x = x_ref[...]
    ab = jnp.dot(x, w_ref[...], preferred_element_type=jnp.float32)
    t_ref[0, :, :HH] = ab[:, :HH]
    t_ref[0, :, HH:] = x[:, :HH]
    t_ref[1, :, :HH] = ab[:, HH:H]
    t_ref[1, :, HH:] = x[:, HH:]
    b_ref[...] = ab[:, H:]


def _prep(state, wcat):
    blk = 1000
    grid = N // blk
    t, b = pl.pallas_call(
        _prep_kernel,
        grid=(grid,),
        in_specs=[
            pl.BlockSpec((blk, H), lambda i: (i, 0)),
            pl.BlockSpec((H, 2 * H), lambda i: (0, 0)),
        ],
        out_specs=[
            pl.BlockSpec((2, blk, H), lambda i: (0, i, 0)),
            pl.BlockSpec((blk, H), lambda i: (i, 0)),
        ],
        out_shape=[
            jax.ShapeDtypeStruct((2, N, H), jnp.float32),
            jax.ShapeDtypeStruct((N, H), jnp.float32),
        ],
    )(state, wcat)
    return t.reshape(2 * N, H), b


def _edge_body(t_hbm, b_hbm, src_hbm, dst_hbm, dist_hbm, out_hbm,
               si_v, di_v, dv_v, gt_v, gb_v, pc_v, zb_v, acc_sh,
               sem1, sem2):
    cid = lax.axis_index("c")
    sid = lax.axis_index("s")
    coff = cid * N
    ooff = cid * NP

    def _zrow(r, carry):
        for c in range(H // L):
            zb_v[r, pl.ds(c * L, L)] = jnp.zeros((L,), jnp.float32)
        return carry
    lax.fori_loop(0, RB, _zrow, 0)

    def _zcopy(bk, carry):
        pltpu.sync_copy(zb_v, acc_sh.at[pl.ds(sid * RPT + bk * RB, RB)])
        return carry
    lax.fori_loop(0, NRB, _zcopy, 0)
    plsc.subcore_barrier()

    def _chunk(k, carry):
        base = sid * EPT + k * CH
        pltpu.sync_copy(src_hbm.at[pl.ds(base, CH)], si_v)
        pltpu.sync_copy(dst_hbm.at[pl.ds(base, CH)], di_v)
        pltpu.sync_copy(dist_hbm.at[pl.ds(base, CH)], dv_v)

        def _off(i, c2):
            s = si_v[pl.ds(i * L, L)]
            si_v[pl.ds(i * L, L)] = s + coff
            return c2
        lax.fori_loop(0, CH // L, _off, 0)

        cp1 = pltpu.async_copy(t_hbm.at[si_v], gt_v, sem1)
        cp2 = pltpu.async_copy(b_hbm.at[di_v], gb_v, sem2)
        cp1.wait()
        cp2.wait()

        def _grp(g, c2):
            dvec = dv_v[pl.ds(g * L, L)]
            for lane in range(L):
                j = g * L + lane
                d = dvec[lane]
                for c in range(HH // L):
                    a = gt_v[j, pl.ds(c * L, L)]
                    b = gb_v[j, pl.ds(cid * HH + c * L, L)]
                    s = gt_v[j, pl.ds(HH + c * L, L)]
                    u = (a + b) * d
                    t = jnp.maximum(u, 0.01 * u)
                    p = jnp.exp(t)
                    pc_v[j, pl.ds(c * L, L)] = p * s
                    pc_v[j, pl.ds(HH + c * L, L)] = p
            return c2
        lax.fori_loop(0, CH // L, _grp, 0)

        pltpu.sync_copy(pc_v, acc_sh.at[di_v], add=True)
        return carry
    lax.fori_loop(0, NCHUNK, _chunk, 0)
    plsc.subcore_barrier()

    def _wb(bk, carry):
        r0 = sid * RPT + bk * RB
        pltpu.sync_copy(acc_sh.at[pl.ds(r0, RB)], zb_v)
        pltpu.sync_copy(zb_v, out_hbm.at[pl.ds(ooff + r0, RB)])
        return carry
    lax.fori_loop(0, NRB, _wb, 0)


def _edge_pass(t, b, src, dst, dist):
    mesh = plsc.VectorSubcoreMesh(
        core_axis_name="c", subcore_axis_name="s", num_cores=NC,
        num_subcores=NS)
    f = pl.kernel(
        _edge_body,
        out_type=jax.ShapeDtypeStruct((2 * NP, H), jnp.float32),
        mesh=mesh,
        scratch_types=[
            pltpu.VMEM((CH,), jnp.int32),
            pltpu.VMEM((CH,), jnp.int32),
            pltpu.VMEM((CH,), jnp.float32),
            pltpu.VMEM((CH, H), jnp.float32),
            pltpu.VMEM((CH, H), jnp.float32),
            pltpu.VMEM((CH, H), jnp.float32),
            pltpu.VMEM((RB, H), jnp.float32),
            pltpu.VMEM_SHARED((NP, H), jnp.float32),
            pltpu.SemaphoreType.DMA,
            pltpu.SemaphoreType.DMA,
        ],
    )
    return f(t, b, src, dst, dist)


def _combine_kernel(lo_ref, hi_ref, o_ref):
    lo = lo_ref[...]
    hi = hi_ref[...]
    n = jnp.concatenate([lo[:, :HH], hi[:, :HH]], axis=1)
    d = jnp.concatenate([lo[:, HH:], hi[:, HH:]], axis=1)
    o_ref[...] = jnp.maximum(n / (d + 1e-16), 0.0)


def _combine(nd):
    blk = 80
    grid = N // blk
    off = NP // blk
    return pl.pallas_call(
        _combine_kernel,
        grid=(grid,),
        in_specs=[
            pl.BlockSpec((blk, H), lambda i: (i, 0)),
            pl.BlockSpec((blk, H), lambda i: (i + off, 0)),
        ],
        out_specs=pl.BlockSpec((blk, H), lambda i: (i, 0)),
        out_shape=jax.ShapeDtypeStruct((N, H), jnp.float32),
    )(nd, nd)


def kernel(state, feature, edge_index, edge_dist, W):
    src = edge_index[0]
    dst = edge_index[1]
    dist = edge_dist.reshape(-1)
    wcat = jnp.concatenate([W[:H, :], W[H:, :]], axis=1)
    t, b = _prep(state, wcat)
    nd = _edge_pass(t, b, src, dst, dist)
    return _combine(nd)


# speedup vs baseline: 3.4821x; 1.2269x over previous
import functools

import jax
import jax.numpy as jnp
from jax import lax
from jax.experimental import pallas as pl
from jax.experimental.pallas import tpu as pltpu
from jax.experimental.pallas import tpu_sc as plsc

N = 10000
NP = 10112
E = 320000
H = 128
HH = H // 2
NC, NS, L = 2, 16, 16
EPT = E // NS
CH = 80
NCHUNK = EPT // CH
RPT = NP // NS
IT = 160
ITROWS = NS * NCHUNK + 8


def _prep_kernel(x_ref, w_ref, t_ref, b_ref):
    x = x_ref[...]
    ab = jnp.dot(x, w_ref[...], preferred_element_type=jnp.float32)
    t_ref[0, :, :HH] = ab[:, :HH]
    t_ref[0, :, HH:] = x[:, :HH]
    t_ref[1, :, :HH] = ab[:, HH:H]
    t_ref[1, :, HH:] = x[:, HH:]
    b_ref[...] = ab[:, H:]


def _prep(state, wcat):
    blk = 1000
    grid = N // blk
    t, b = pl.pallas_call(
        _prep_kernel,
        grid=(grid,),
        in_specs=[
            pl.BlockSpec((blk, H), lambda i: (i, 0)),
            pl.BlockSpec((H, 2 * H), lambda i: (0, 0)),
        ],
        out_specs=[
            pl.BlockSpec((2, blk, H), lambda i: (0, i, 0)),
            pl.BlockSpec((blk, H), lambda i: (i, 0)),
        ],
        out_shape=[
            jax.ShapeDtypeStruct((2, N, H), jnp.float32),
            jax.ShapeDtypeStruct((N, H), jnp.float32),
        ],
    )(state, wcat)
    return t.reshape(2 * N, H), b


def _edge_body(t_hbm, b_hbm, itab_hbm, dtab_hbm, out_hbm,
               it0, it1, dv0, dv1, di0, di1, gt0, gt1, pc0, pc1, acc_sh,
               semi0, semi1, semg0, semg1, sems0, sems1):
    it = (it0, it1)
    dv = (dv0, dv1)
    di = (di0, di1)
    gt = (gt0, gt1)
    pc = (pc0, pc1)
    semi = (semi0, semi1)
    semg = (semg0, semg1)
    sems = (sems0, sems1)

    cid = lax.axis_index("c")
    sid = lax.axis_index("s")
    coff = cid * N
    ooff = cid * NP

    def _zrow(r, carry):
        for c in range(H // L):
            gt0[r, pl.ds(c * L, L)] = jnp.zeros((L,), jnp.float32)
        return carry
    lax.fori_loop(0, CH, _zrow, 0)
    r0 = sid * RPT
    for i in range(RPT // CH):
        pltpu.sync_copy(gt0, acc_sh.at[pl.ds(r0 + i * CH, CH)])
    rem = RPT % CH
    pltpu.sync_copy(gt0.at[pl.ds(0, rem)],
                    acc_sh.at[pl.ds(r0 + RPT - rem, rem)])
    plsc.subcore_barrier()

    def issue_idx(k, b):
        g = sid * NCHUNK + k
        pltpu.async_copy(itab_hbm.at[pl.ds(g * IT, IT)], it[b], semi[b])
        pltpu.async_copy(dtab_hbm.at[pl.ds(g * CH, CH)], dv[b], semi[b])

    def wait_idx(k, b):
        g = sid * NCHUNK + k
        pltpu.make_async_copy(
            itab_hbm.at[pl.ds(g * IT, IT)], it[b], semi[b]).wait()
        pltpu.make_async_copy(
            dtab_hbm.at[pl.ds(g * CH, CH)], dv[b], semi[b]).wait()

    def fix_and_gather(k, b):
        wait_idx(k, b)
        for i in range(CH // L):
            s = it[b][pl.ds(i * L, L)]
            it[b][pl.ds(i * L, L)] = s + coff
            di[b][pl.ds(i * L, L)] = it[b][pl.ds(CH + i * L, L)]
        pltpu.async_copy(t_hbm.at[it[b].at[pl.ds(0, CH)]], gt[b], semg[b])
        pltpu.async_copy(b_hbm.at[di[b]], pc[b], semg[b])

    def wait_gathers(b):
        pltpu.make_async_copy(
            t_hbm.at[it[b].at[pl.ds(0, CH)]], gt[b], semg[b]).wait()
        pltpu.make_async_copy(b_hbm.at[di[b]], pc[b], semg[b]).wait()

    def compute(b):
        def _grp(g, c2):
            dvec = dv[b][pl.ds(g * L, L)]
            for lane in range(L):
                j = g * L + lane
                d = dvec[lane]
                for c in range(HH // L):
                    a = gt[b][j, pl.ds(c * L, L)]
                    bb = pc[b][j, pl.ds(cid * HH + c * L, L)]
                    s = gt[b][j, pl.ds(HH + c * L, L)]
                    u = (a + bb) * d
                    t = jnp.maximum(u, 0.01 * u)
                    p = jnp.exp(t)
                    pc[b][j, pl.ds(c * L, L)] = p * s
                    pc[b][j, pl.ds(HH + c * L, L)] = p
            return c2
        lax.fori_loop(0, CH // L, _grp, 0)

    def scatter(b):
        pltpu.sync_copy(pc[b], acc_sh.at[di[b]], add=True)

    def wait_scatter(b):
        pass

    issue_idx(0, 0)
    issue_idx(1, 1)
    fix_and_gather(0, 0)

    def _step(ko, carry):
        for b in range(2):
            k = 2 * ko + b
            fix_and_gather(k + 1, 1 - b)
            wait_gathers(b)
            compute(b)
            issue_idx(k + 2, b)
            scatter(b)
        return carry
    lax.fori_loop(0, NCHUNK // 2, _step, 0)

    wait_gathers(0)
    wait_idx(NCHUNK + 1, 1)
    plsc.subcore_barrier()

    pltpu.sync_copy(acc_sh.at[pl.ds(sid * RPT, RPT)],
                    out_hbm.at[pl.ds(ooff + sid * RPT, RPT)])


def _edge_pass(t, b, itab, dtab):
    mesh = plsc.VectorSubcoreMesh(
        core_axis_name="c", subcore_axis_name="s", num_cores=NC,
        num_subcores=NS)
    dbl = lambda ty: [ty, ty]
    f = pl.kernel(
        _edge_body,
        out_type=jax.ShapeDtypeStruct((2 * NP, H), jnp.float32),
        mesh=mesh,
        scratch_types=[
            *dbl(pltpu.VMEM((IT,), jnp.int32)),
            *dbl(pltpu.VMEM((CH,), jnp.float32)),
            *dbl(pltpu.VMEM((CH,), jnp.int32)),
            *dbl(pltpu.VMEM((CH, H), jnp.float32)),
            *dbl(pltpu.VMEM((CH, H), jnp.float32)),
            pltpu.VMEM_SHARED((NP, H), jnp.float32),
            *[pltpu.SemaphoreType.DMA] * 6,
        ],
    )
    return f(t, b, itab, dtab)


def _combine_kernel(lo_ref, hi_ref, o_ref):
    lo = lo_ref[...]
    hi = hi_ref[...]
    n = jnp.concatenate([lo[:, :HH], hi[:, :HH]], axis=1)
    d = jnp.concatenate([lo[:, HH:], hi[:, HH:]], axis=1)
    o_ref[...] = jnp.maximum(n / (d + 1e-16), 0.0)


def _combine(nd):
    blk = 16
    grid = N // blk
    off = NP // blk
    return pl.pallas_call(
        _combine_kernel,
        grid=(grid,),
        in_specs=[
            pl.BlockSpec((blk, H), lambda i: (i, 0)),
            pl.BlockSpec((blk, H), lambda i: (i + off, 0)),
        ],
        out_specs=pl.BlockSpec((blk, H), lambda i: (i, 0)),
        out_shape=jax.ShapeDtypeStruct((N, H), jnp.float32),
    )(nd, nd)


def kernel(state, feature, edge_index, edge_dist, W):
    src = edge_index[0]
    dst = edge_index[1]
    dist = edge_dist.reshape(-1)
    itab = jnp.concatenate(
        [src.reshape(-1, CH), dst.reshape(-1, CH)], axis=1)
    itab = jnp.concatenate(
        [itab, jnp.zeros((ITROWS - E // CH, IT), jnp.int32)]).reshape(-1)
    dtab = jnp.concatenate([dist, jnp.zeros((ITROWS * CH - E,), jnp.float32)])
    wcat = jnp.concatenate([W[:H, :], W[H:, :]], axis=1)
    t, b = _prep(state, wcat)
    nd = _edge_pass(t, b, itab, dtab)
    return _combine(nd)


# speedup vs baseline: 3.4840x; 1.0005x over previous
import functools

import jax
import jax.numpy as jnp
from jax import lax
from jax.experimental import pallas as pl
from jax.experimental.pallas import tpu as pltpu
from jax.experimental.pallas import tpu_sc as plsc

N = 10000
NP = 10112
E = 320000
H = 128
HH = H // 2
NC, NS, L = 2, 16, 16
EPT = E // NS
CH = 80
NCHUNK = EPT // CH
RPT = NP // NS
IT = 160
ITROWS = NS * NCHUNK + 8


def _prep_kernel(x_ref, w_ref, t_ref, b_ref):
    x = x_ref[...]
    ab = jnp.dot(x, w_ref[...], preferred_element_type=jnp.float32)
    t_ref[0, :, :HH] = ab[:, :HH]
    t_ref[0, :, HH:] = x[:, :HH]
    t_ref[1, :, :HH] = ab[:, HH:H]
    t_ref[1, :, HH:] = x[:, HH:]
    b_ref[...] = ab[:, H:]


def _prep(state, wcat):
    blk = 1000
    grid = N // blk
    t, b = pl.pallas_call(
        _prep_kernel,
        grid=(grid,),
        in_specs=[
            pl.BlockSpec((blk, H), lambda i: (i, 0)),
            pl.BlockSpec((H, 2 * H), lambda i: (0, 0)),
        ],
        out_specs=[
            pl.BlockSpec((2, blk, H), lambda i: (0, i, 0)),
            pl.BlockSpec((blk, H), lambda i: (i, 0)),
        ],
        out_shape=[
            jax.ShapeDtypeStruct((2, N, H), jnp.float32),
            jax.ShapeDtypeStruct((N, H), jnp.float32),
        ],
    )(state, wcat)
    return t.reshape(2 * N, H), b


def _edge_body(t_hbm, b_hbm, itab_hbm, dtab_hbm, out_hbm,
               it0, it1, dv0, dv1, di0, di1, gt0, gt1, pc0, pc1, acc_sh,
               semi0, semi1, semg0, semg1, sems0, sems1):
    it = (it0, it1)
    dv = (dv0, dv1)
    di = (di0, di1)
    gt = (gt0, gt1)
    pc = (pc0, pc1)
    semi = (semi0, semi1)
    semg = (semg0, semg1)
    sems = (sems0, sems1)

    cid = lax.axis_index("c")
    sid = lax.axis_index("s")
    coff = cid * N
    ooff = cid * NP

    def _zrow(r, carry):
        for c in range(H // L):
            gt0[r, pl.ds(c * L, L)] = jnp.zeros((L,), jnp.float32)
        return carry
    lax.fori_loop(0, CH, _zrow, 0)
    r0 = sid * RPT
    for i in range(RPT // CH):
        pltpu.sync_copy(gt0, acc_sh.at[pl.ds(r0 + i * CH, CH)])
    rem = RPT % CH
    pltpu.sync_copy(gt0.at[pl.ds(0, rem)],
                    acc_sh.at[pl.ds(r0 + RPT - rem, rem)])
    plsc.subcore_barrier()

    def issue_idx(k, b):
        g = sid * NCHUNK + k
        pltpu.async_copy(itab_hbm.at[pl.ds(g * IT, IT)], it[b], semi[b])
        pltpu.async_copy(dtab_hbm.at[pl.ds(g * CH, CH)], dv[b], semi[b])

    def wait_idx(k, b):
        g = sid * NCHUNK + k
        pltpu.make_async_copy(
            itab_hbm.at[pl.ds(g * IT, IT)], it[b], semi[b]).wait()
        pltpu.make_async_copy(
            dtab_hbm.at[pl.ds(g * CH, CH)], dv[b], semi[b]).wait()

    def fix_and_gather(k, b):
        wait_idx(k, b)
        for i in range(CH // L):
            s = it[b][pl.ds(i * L, L)]
            it[b][pl.ds(i * L, L)] = s + coff
            di[b][pl.ds(i * L, L)] = it[b][pl.ds(CH + i * L, L)]
        pltpu.async_copy(t_hbm.at[it[b].at[pl.ds(0, CH)]], gt[b], semg[b])
        pltpu.async_copy(b_hbm.at[di[b]], pc[b], semg[b])

    def wait_gathers(b):
        pltpu.make_async_copy(
            t_hbm.at[it[b].at[pl.ds(0, CH)]], gt[b], semg[b]).wait()
        pltpu.make_async_copy(b_hbm.at[di[b]], pc[b], semg[b]).wait()

    def compute(b):
        def _grp(g, c2):
            dvec = dv[b][pl.ds(g * L, L)]
            for lane in range(L):
                j = g * L + lane
                d = dvec[lane]
                for c in range(HH // L):
                    a = gt[b][j, pl.ds(c * L, L)]
                    bb = pc[b][j, pl.ds(cid * HH + c * L, L)]
                    s = gt[b][j, pl.ds(HH + c * L, L)]
                    u = (a + bb) * d
                    t = jnp.maximum(u, 0.01 * u)
                    p = jnp.exp(t)
                    pc[b][j, pl.ds(c * L, L)] = p * s
                    pc[b][j, pl.ds(HH + c * L, L)] = p
            return c2
        lax.fori_loop(0, CH // L, _grp, 0)

    def scatter(b):
        pltpu.async_copy(pc[b], acc_sh.at[di[b]], sems[b], add=True)

    def wait_scatter(b):
        pltpu.make_async_copy(pc[b], acc_sh.at[di[b]], sems[b]).wait()

    issue_idx(0, 0)
    issue_idx(1, 1)
    fix_and_gather(0, 0)

    def _step(ko, carry):
        for b in range(2):
            k = 2 * ko + b
            @pl.when(k >= 1)
            def _():
                wait_scatter(1 - b)
            fix_and_gather(k + 1, 1 - b)
            wait_gathers(b)
            compute(b)
            issue_idx(k + 2, b)
            scatter(b)
        return carry
    lax.fori_loop(0, NCHUNK // 2, _step, 0)

    wait_scatter(1)
    wait_gathers(0)
    wait_idx(NCHUNK + 1, 1)
    plsc.subcore_barrier()

    pltpu.sync_copy(acc_sh.at[pl.ds(sid * RPT, RPT)],
                    out_hbm.at[pl.ds(ooff + sid * RPT, RPT)])


def _edge_pass(t, b, itab, dtab):
    mesh = plsc.VectorSubcoreMesh(
        core_axis_name="c", subcore_axis_name="s", num_cores=NC,
        num_subcores=NS)
    dbl = lambda ty: [ty, ty]
    f = pl.kernel(
        _edge_body,
        out_type=jax.ShapeDtypeStruct((2 * NP, H), jnp.float32),
        mesh=mesh,
        scratch_types=[
            *dbl(pltpu.VMEM((IT,), jnp.int32)),
            *dbl(pltpu.VMEM((CH,), jnp.float32)),
            *dbl(pltpu.VMEM((CH,), jnp.int32)),
            *dbl(pltpu.VMEM((CH, H), jnp.float32)),
            *dbl(pltpu.VMEM((CH, H), jnp.float32)),
            pltpu.VMEM_SHARED((NP, H), jnp.float32),
            *[pltpu.SemaphoreType.DMA] * 6,
        ],
    )
    return f(t, b, itab, dtab)


def _combine_kernel(lo_ref, hi_ref, o_ref):
    lo = lo_ref[...]
    hi = hi_ref[...]
    n = jnp.concatenate([lo[:, :HH], hi[:, :HH]], axis=1)
    d = jnp.concatenate([lo[:, HH:], hi[:, HH:]], axis=1)
    o_ref[...] = jnp.maximum(n / (d + 1e-16), 0.0)


def _combine(nd):
    blk = 16
    grid = N // blk
    off = NP // blk
    return pl.pallas_call(
        _combine_kernel,
        grid=(grid,),
        in_specs=[
            pl.BlockSpec((blk, H), lambda i: (i, 0)),
            pl.BlockSpec((blk, H), lambda i: (i + off, 0)),
        ],
        out_specs=pl.BlockSpec((blk, H), lambda i: (i, 0)),
        out_shape=jax.ShapeDtypeStruct((N, H), jnp.float32),
    )(nd, nd)


def kernel(state, feature, edge_index, edge_dist, W):
    src = edge_index[0]
    dst = edge_index[1]
    dist = edge_dist.reshape(-1)
    itab = jnp.concatenate(
        [src.reshape(-1, CH), dst.reshape(-1, CH)], axis=1)
    itab = jnp.concatenate(
        [itab, jnp.zeros((ITROWS - E // CH, IT), jnp.int32)]).reshape(-1)
    dtab = jnp.concatenate([dist, jnp.zeros((ITROWS * CH - E,), jnp.float32)])
    wcat = jnp.concatenate([W[:H, :], W[H:, :]], axis=1)
    t, b = _prep(state, wcat)
    nd = _edge_pass(t, b, itab, dtab)
    return _combine(nd)


# speedup vs baseline: 5.7177x; 1.6412x over previous
import functools

import jax
import jax.numpy as jnp
from jax import lax
from jax.experimental import pallas as pl
from jax.experimental.pallas import tpu as pltpu
from jax.experimental.pallas import tpu_sc as plsc

N = 10000
NP = 10112
E = 320000
H = 128
HH = H // 2
NC, NS, L = 2, 16, 16
EPT = E // NS
CH = 64
NCHUNK = 314
EPTP = NCHUNK * CH
RPT = NP // NS
IT = 2 * CH
ITROWS = NS * NCHUNK + 8


def _prep_kernel(x_ref, w_ref, t_ref, b_ref):
    x = x_ref[...]
    ab = jnp.dot(x, w_ref[...], preferred_element_type=jnp.float32)
    t_ref[0, :, :HH] = ab[:, :HH]
    t_ref[0, :, HH:] = x[:, :HH]
    t_ref[1, :, :HH] = ab[:, HH:H]
    t_ref[1, :, HH:] = x[:, HH:]
    b_ref[...] = ab[:, H:]


def _prep(state, wcat):
    blk = 1000
    grid = N // blk
    t, b = pl.pallas_call(
        _prep_kernel,
        grid=(grid,),
        in_specs=[
            pl.BlockSpec((blk, H), lambda i: (i, 0)),
            pl.BlockSpec((H, 2 * H), lambda i: (0, 0)),
        ],
        out_specs=[
            pl.BlockSpec((2, blk, H), lambda i: (0, i, 0)),
            pl.BlockSpec((blk, H), lambda i: (i, 0)),
        ],
        out_shape=[
            jax.ShapeDtypeStruct((2, N, H), jnp.float32),
            jax.ShapeDtypeStruct((N, H), jnp.float32),
        ],
    )(state, wcat)
    return t.reshape(2 * N, H), b


def _edge_body(t_hbm, b_hbm, itab_hbm, dtab_hbm, out_hbm,
               it0, it1, dv0, dv1, di0, di1, gt0, gt1, gb0, gb1, pc0, pc1,
               acc_sh, semi0, semi1, semg0, semg1, sems0, sems1):
    it = (it0, it1)
    dv = (dv0, dv1)
    di = (di0, di1)
    gt = (gt0, gt1)
    gb = (gb0, gb1)
    pc = (pc0, pc1)
    semi = (semi0, semi1)
    semg = (semg0, semg1)
    sems = (sems0, sems1)

    cid = lax.axis_index("c")
    sid = lax.axis_index("s")
    coff = cid * N
    ooff = cid * NP

    def _zrow(r, carry):
        for c in range(H // L):
            gt0[r, pl.ds(c * L, L)] = jnp.zeros((L,), jnp.float32)
        return carry
    lax.fori_loop(0, CH, _zrow, 0)
    r0 = sid * RPT
    for i in range(RPT // CH):
        pltpu.sync_copy(gt0, acc_sh.at[pl.ds(r0 + i * CH, CH)])
    rem = RPT % CH
    pltpu.sync_copy(gt0.at[pl.ds(0, rem)],
                    acc_sh.at[pl.ds(r0 + RPT - rem, rem)])
    plsc.subcore_barrier()

    def issue_idx(k, b):
        g = sid * NCHUNK + k
        pltpu.async_copy(itab_hbm.at[pl.ds(g * IT, IT)], it[b], semi[b])
        pltpu.async_copy(dtab_hbm.at[pl.ds(g * CH, CH)], dv[b], semi[b])

    def wait_idx(k, b):
        g = sid * NCHUNK + k
        pltpu.make_async_copy(
            itab_hbm.at[pl.ds(g * IT, IT)], it[b], semi[b]).wait()
        pltpu.make_async_copy(
            dtab_hbm.at[pl.ds(g * CH, CH)], dv[b], semi[b]).wait()

    def fix_and_gather(k, b):
        wait_idx(k, b)
        for i in range(CH // L):
            s = it[b][pl.ds(i * L, L)]
            it[b][pl.ds(i * L, L)] = s + coff
            di[b][pl.ds(i * L, L)] = it[b][pl.ds(CH + i * L, L)]
        pltpu.async_copy(t_hbm.at[it[b].at[pl.ds(0, CH)]], gt[b], semg[b])
        pltpu.async_copy(b_hbm.at[di[b]], gb[b], semg[b])

    def wait_gathers(b):
        pltpu.make_async_copy(
            t_hbm.at[it[b].at[pl.ds(0, CH)]], gt[b], semg[b]).wait()
        pltpu.make_async_copy(b_hbm.at[di[b]], gb[b], semg[b]).wait()

    def compute(b):
        nc = HH // L

        def _grp(g):
            dvec = dv[b][pl.ds(g * L, L)]
            for lane in range(L):
                j = g * L + lane
                d = dvec[lane]
                aa = [gt[b][j, pl.ds(c * L, L)] for c in range(nc)]
                bb = [gb[b][j, pl.ds(cid * HH + c * L, L)] for c in range(nc)]
                ss = [gt[b][j, pl.ds(HH + c * L, L)] for c in range(nc)]
                us = [(aa[c] + bb[c]) * d for c in range(nc)]
                ts = [jnp.maximum(u, 0.01 * u) for u in us]
                ps = [jnp.exp(t) for t in ts]
                for c in range(nc):
                    pc[b][j, pl.ds(c * L, L)] = ps[c] * ss[c]
                    pc[b][j, pl.ds(HH + c * L, L)] = ps[c]
        plsc.parallel_loop(0, CH // L)(_grp)

    def scatter(b):
        pltpu.async_copy(pc[b], acc_sh.at[di[b]], sems[b], add=True)

    def wait_scatter(b):
        pltpu.make_async_copy(pc[b], acc_sh.at[di[b]], sems[b]).wait()

    issue_idx(0, 0)
    issue_idx(1, 1)
    fix_and_gather(0, 0)

    def _step(ko, carry):
        for b in range(2):
            k = 2 * ko + b
            @pl.when(k >= 1)
            def _():
                wait_scatter(1 - b)
            fix_and_gather(k + 1, 1 - b)
            wait_gathers(b)
            compute(b)
            issue_idx(k + 2, b)
            scatter(b)
        return carry
    lax.fori_loop(0, NCHUNK // 2, _step, 0)

    wait_scatter(1)
    wait_gathers(0)
    wait_idx(NCHUNK + 1, 1)
    plsc.subcore_barrier()

    pltpu.sync_copy(acc_sh.at[pl.ds(sid * RPT, RPT)],
                    out_hbm.at[pl.ds(ooff + sid * RPT, RPT)])


def _edge_pass(t, b, itab, dtab):
    mesh = plsc.VectorSubcoreMesh(
        core_axis_name="c", subcore_axis_name="s", num_cores=NC,
        num_subcores=NS)
    dbl = lambda ty: [ty, ty]
    f = pl.kernel(
        _edge_body,
        out_type=jax.ShapeDtypeStruct((2 * NP, H), jnp.float32),
        mesh=mesh,
        scratch_types=[
            *dbl(pltpu.VMEM((IT,), jnp.int32)),
            *dbl(pltpu.VMEM((CH,), jnp.float32)),
            *dbl(pltpu.VMEM((CH,), jnp.int32)),
            *dbl(pltpu.VMEM((CH, H), jnp.float32)),
            *dbl(pltpu.VMEM((CH, H), jnp.float32)),
            *dbl(pltpu.VMEM((CH, H), jnp.float32)),
            pltpu.VMEM_SHARED((NP, H), jnp.float32),
            *[pltpu.SemaphoreType.DMA] * 6,
        ],
    )
    return f(t, b, itab, dtab)


def _combine_kernel(lo_ref, hi_ref, o_ref):
    lo = lo_ref[...]
    hi = hi_ref[...]
    n = jnp.concatenate([lo[:, :HH], hi[:, :HH]], axis=1)
    d = jnp.concatenate([lo[:, HH:], hi[:, HH:]], axis=1)
    o_ref[...] = jnp.maximum(n / (d + 1e-16), 0.0)


def _combine(nd):
    blk = 16
    grid = N // blk
    off = NP // blk
    return pl.pallas_call(
        _combine_kernel,
        grid=(grid,),
        in_specs=[
            pl.BlockSpec((blk, H), lambda i: (i, 0)),
            pl.BlockSpec((blk, H), lambda i: (i + off, 0)),
        ],
        out_specs=pl.BlockSpec((blk, H), lambda i: (i, 0)),
        out_shape=jax.ShapeDtypeStruct((N, H), jnp.float32),
    )(nd, nd)


def kernel(state, feature, edge_index, edge_dist, W):
    src = edge_index[0]
    dst = edge_index[1]
    dist = edge_dist.reshape(-1)
    pad = EPTP - EPT
    srcp = jnp.pad(src.reshape(NS, EPT), ((0, 0), (0, pad)))
    dstp = jnp.pad(dst.reshape(NS, EPT), ((0, 0), (0, pad)),
                   constant_values=N)
    distp = jnp.pad(dist.reshape(NS, EPT), ((0, 0), (0, pad)))
    itab = jnp.concatenate(
        [srcp.reshape(-1, CH), dstp.reshape(-1, CH)], axis=1)
    itab = jnp.concatenate(
        [itab, jnp.zeros((8, IT), jnp.int32)]).reshape(-1)
    dtab = jnp.concatenate(
        [distp.reshape(-1), jnp.zeros((8 * CH,), jnp.float32)])
    wcat = jnp.concatenate([W[:H, :], W[H:, :]], axis=1)
    t, b = _prep(state, wcat)
    nd = _edge_pass(t, b, itab, dtab)
    return _combine(nd)
